# baseline (speedup 1.0000x reference)
import functools
import jax
import jax.numpy as jnp
from jax import lax
from jax.experimental import pallas as pl
from jax.experimental.pallas import tpu as pltpu, tpu_sc as plsc

NUM_GRAPHS = 10
NEG = float("-inf")


def _make_phase0(E, NW, NBPW, RB, CAPW, CAPB, CE):
    NPW = RB * NBPW
    NBUK = NW * NBPW

    def body(src_hbm, dst_hbm, bsrc_hbm, bdl_hbm, bcnt_hbm,
             sbuf, dbuf, tsrc, tdst, stg_s, stg_d, cbuf):
        wid = lax.axis_index("s") * 2 + lax.axis_index("c")
        lo = wid * NPW
        hi = lo + NPW
        lane = lax.iota(jnp.int32, 16)

        def chunk_body(ch, off):
            pltpu.sync_copy(src_hbm.at[pl.ds(ch * CE, CE)], sbuf)
            pltpu.sync_copy(dst_hbm.at[pl.ds(ch * CE, CE)], dbuf)

            def scan_body(j, off):
                d = dbuf[pl.ds(j * 16, 16)]
                sv = sbuf[pl.ds(j * 16, 16)]
                m = (d >= lo) & (d < hi)
                csum = plsc.cumsum(m.astype(jnp.int32))
                pos = off + csum - 1
                m2 = m & (pos < CAPW)
                plsc.store_scatter(tsrc, [pos], sv, mask=m2)
                plsc.store_scatter(tdst, [pos], d, mask=m2)
                return off + csum[15]

            return lax.fori_loop(0, CE // 16, scan_body, off)

        off = lax.fori_loop(0, E // CE, chunk_body, 0)

        def bucket_body(b, _):
            blo = lo + b * RB

            def ms(j, _):
                stg_s[pl.ds(j * 16, 16)] = jnp.zeros((16,), jnp.int32)
                stg_d[pl.ds(j * 16, 16)] = jnp.full((16,), RB, jnp.int32)
                return 0

            lax.fori_loop(0, CAPB // 16, ms, 0)

            ngrp = (off + 15) // 16

            def fil(j, boff):
                d = tdst[pl.ds(j * 16, 16)]
                sv = tsrc[pl.ds(j * 16, 16)]
                valid = (j * 16 + lane) < off
                m = valid & (d >= blo) & (d < blo + RB)
                csum = plsc.cumsum(m.astype(jnp.int32))
                pos = boff + csum - 1
                m2 = m & (pos < CAPB)
                plsc.store_scatter(stg_s, [pos], sv, mask=m2)
                plsc.store_scatter(stg_d, [pos], d - blo, mask=m2)
                return boff + csum[15]

            boff = lax.fori_loop(0, ngrp, fil, 0)
            bucket = wid * NBPW + b
            pltpu.sync_copy(stg_s, bsrc_hbm.at[bucket])
            pltpu.sync_copy(stg_d, bdl_hbm.at[bucket])
            cbuf[pl.ds(0, 16)] = jnp.full((16,), boff, jnp.int32)
            pltpu.sync_copy(cbuf, bcnt_hbm.at[bucket])
            return 0

        lax.fori_loop(0, NBPW, bucket_body, 0)

    mesh = plsc.VectorSubcoreMesh(core_axis_name="c", subcore_axis_name="s")
    return pl.kernel(
        body,
        out_type=(
            jax.ShapeDtypeStruct((NBUK, CAPB), jnp.int32),
            jax.ShapeDtypeStruct((NBUK, CAPB), jnp.int32),
            jax.ShapeDtypeStruct((NBUK, 16), jnp.int32),
        ),
        mesh=mesh,
        compiler_params=pltpu.CompilerParams(
            use_tc_tiling_on_sc=False, needs_layout_passes=False),
        scratch_types=[
            pltpu.VMEM((CE,), jnp.int32),
            pltpu.VMEM((CE,), jnp.int32),
            pltpu.VMEM((CAPW,), jnp.int32),
            pltpu.VMEM((CAPW,), jnp.int32),
            pltpu.VMEM((CAPB,), jnp.int32),
            pltpu.VMEM((CAPB,), jnp.int32),
            pltpu.VMEM((16,), jnp.int32),
        ],
    )


def _make_agg(NW, NBPW, RB, CAPB, dout, G, SR):
    BPC = min(512 // dout, NBPW)
    NCH = NBPW // BPC
    ACC_R = BPC * (RB + 1)
    NPW = RB * NBPW
    NPAD = NW * NPW

    def body(h_hbm, bsrc_hbm, bdl_hbm, bcnt_hbm, agg_hbm,
             idx_v, dl_v, rows_v, acc, cbuf, strip, sem):
        wid = lax.axis_index("s") * 2 + lax.axis_index("c")

        def chunk_body(ch, _):
            def ini(j, _):
                for f in range(dout // 16):
                    acc[j, pl.ds(f * 16, 16)] = jnp.full((16,), NEG, jnp.float32)
                return 0

            lax.fori_loop(0, ACC_R, ini, 0)

            def bucket_body(b2, _):
                bucket = wid * NBPW + ch * BPC + b2
                pltpu.sync_copy(bcnt_hbm.at[bucket], cbuf)
                cnt = cbuf[pl.ds(0, 16)][0]
                nbat = (cnt + G - 1) // G
                arow0 = b2 * (RB + 1)

                def bat(k, _):
                    pltpu.sync_copy(bsrc_hbm.at[bucket, pl.ds(k * G, G)], idx_v)
                    pltpu.sync_copy(bdl_hbm.at[bucket, pl.ds(k * G, G)], dl_v)
                    pltpu.async_copy(h_hbm.at[idx_v], rows_v, sem).wait()

                    def grp(j, _):
                        dlv = dl_v[pl.ds(j * 16, 16)]
                        for l in range(16):
                            arow = arow0 + dlv[l]
                            erow = j * 16 + l
                            for f in range(dout // 16):
                                sl = pl.ds(f * 16, 16)
                                acc[arow, sl] = jnp.maximum(
                                    acc[arow, sl], rows_v[erow, sl])
                        return 0

                    lax.fori_loop(0, G // 16, grp, 0)
                    return 0

                lax.fori_loop(0, nbat, bat, 0)
                return 0

            lax.fori_loop(0, BPC, bucket_body, 0)

            nstr = RB // SR

            def wb(st, _):
                b2 = st // nstr
                s2 = st % nstr
                ar0 = b2 * (RB + 1) + s2 * SR
                for r in range(SR):
                    for f in range(dout // 16):
                        sl = pl.ds(f * 16, 16)
                        v = acc[ar0 + r, sl]
                        strip[r, sl] = jnp.where(v == NEG,
                                                 jnp.zeros((16,), jnp.float32), v)
                node0 = (wid * NBPW + ch * BPC + b2) * RB + s2 * SR
                pltpu.sync_copy(strip, agg_hbm.at[pl.ds(node0, SR)])
                return 0

            lax.fori_loop(0, BPC * nstr, wb, 0)
            return 0

        lax.fori_loop(0, NCH, chunk_body, 0)

    mesh = plsc.VectorSubcoreMesh(core_axis_name="c", subcore_axis_name="s")
    return pl.kernel(
        body,
        out_type=jax.ShapeDtypeStruct((NPAD, dout), jnp.float32),
        mesh=mesh,
        compiler_params=pltpu.CompilerParams(
            use_tc_tiling_on_sc=False, needs_layout_passes=False),
        scratch_types=[
            pltpu.VMEM((G,), jnp.int32),
            pltpu.VMEM((G,), jnp.int32),
            pltpu.VMEM((G, dout), jnp.float32),
            pltpu.VMEM((ACC_R, dout), jnp.float32),
            pltpu.VMEM((16,), jnp.int32),
            pltpu.VMEM((SR, dout), jnp.float32),
            pltpu.SemaphoreType.DMA,
        ],
    )


def _mlp1_body(pos_ref, w1_ref, b1_ref, w2_ref, b2_ref, h_ref):
    h1 = jnp.maximum(
        jnp.dot(pos_ref[...], w1_ref[...],
                preferred_element_type=jnp.float32) + b1_ref[...], 0.0)
    h_ref[...] = jnp.dot(h1, w2_ref[...],
                         preferred_element_type=jnp.float32) + b2_ref[...]


def _mlp2_body(pos_ref, agg_ref, w1p_ref, w1a_ref, b1_ref, w2_ref, b2_ref,
               h_ref):
    pre = (jnp.dot(pos_ref[...], w1p_ref[...],
                   preferred_element_type=jnp.float32)
           + jnp.dot(agg_ref[...], w1a_ref[...],
                     preferred_element_type=jnp.float32) + b1_ref[...])
    h1 = jnp.maximum(pre, 0.0)
    h_ref[...] = jnp.dot(h1, w2_ref[...],
                         preferred_element_type=jnp.float32) + b2_ref[...]


def _run_mlp(x_parts, weights, N, BN, dout):
    grid = N // BN
    full = lambda shape: pl.BlockSpec(shape, lambda i: tuple(0 for _ in shape))
    if len(x_parts) == 1:
        (posx,) = x_parts
        W1, b1, W2, b2 = weights
        return pl.pallas_call(
            _mlp1_body,
            grid=(grid,),
            in_specs=[
                pl.BlockSpec((BN, posx.shape[1]), lambda i: (i, 0)),
                full(W1.shape), full(b1.shape), full(W2.shape), full(b2.shape),
            ],
            out_specs=pl.BlockSpec((BN, dout), lambda i: (i, 0)),
            out_shape=jax.ShapeDtypeStruct((N, dout), jnp.float32),
        )(posx, W1, b1, W2, b2)
    posx, aggx = x_parts
    W1p, W1a, b1, W2, b2 = weights
    return pl.pallas_call(
        _mlp2_body,
        grid=(grid,),
        in_specs=[
            pl.BlockSpec((BN, posx.shape[1]), lambda i: (i, 0)),
            pl.BlockSpec((BN, aggx.shape[1]), lambda i: (i, 0)),
            full(W1p.shape), full(W1a.shape), full(b1.shape),
            full(W2.shape), full(b2.shape),
        ],
        out_specs=pl.BlockSpec((BN, dout), lambda i: (i, 0)),
        out_shape=jax.ShapeDtypeStruct((N, dout), jnp.float32),
    )(posx, aggx, W1p, W1a, b1, W2, b2)


def _final_body(x_ref, oh_ref, wf1_ref, bf1_ref, wf2_ref, bf2_ref, out_ref,
                gacc):
    step = pl.program_id(0)
    nsteps = pl.num_programs(0)

    @pl.when(step == 0)
    def _():
        gacc[...] = jnp.full_like(gacc[...], NEG)

    x = x_ref[...]
    oh = oh_ref[...]
    cur = gacc[...]
    rows = []
    for g in range(NUM_GRAPHS):
        mask = oh[:, g:g + 1] > 0.5
        mx = jnp.max(jnp.where(mask, x, NEG), axis=0)
        rows.append(mx)
    for g in range(NUM_GRAPHS, 16):
        rows.append(jnp.full((x.shape[1],), NEG, jnp.float32))
    upd = jnp.stack(rows, axis=0)
    gacc[...] = jnp.maximum(cur, upd)

    @pl.when(step == nsteps - 1)
    def _():
        gmat = gacc[...][:NUM_GRAPHS]
        gmat = jnp.where(jnp.isfinite(gmat), gmat, 0.0)
        h1 = jnp.maximum(
            jnp.dot(gmat, wf1_ref[...], preferred_element_type=jnp.float32)
            + bf1_ref[...], 0.0)
        logits = jnp.dot(h1, wf2_ref[...],
                         preferred_element_type=jnp.float32) + bf2_ref[...]
        m = jnp.max(logits, axis=1, keepdims=True)
        e = jnp.exp(logits - m)
        s = jnp.sum(e, axis=1, keepdims=True)
        out_ref[...] = logits - m - jnp.log(s)


def _run_final(x4, onehot, fc, N, BN):
    Wf1, bf1, Wf2, bf2 = fc
    grid = N // BN
    full = lambda shape: pl.BlockSpec(shape, lambda i: tuple(0 for _ in shape))
    return pl.pallas_call(
        _final_body,
        grid=(grid,),
        in_specs=[
            pl.BlockSpec((BN, x4.shape[1]), lambda i: (i, 0)),
            pl.BlockSpec((BN, onehot.shape[1]), lambda i: (i, 0)),
            full(Wf1.shape), full(bf1.shape), full(Wf2.shape), full(bf2.shape),
        ],
        out_specs=pl.BlockSpec((NUM_GRAPHS, NUM_GRAPHS), lambda i: (0, 0)),
        out_shape=jax.ShapeDtypeStruct((NUM_GRAPHS, NUM_GRAPHS), jnp.float32),
        scratch_shapes=[pltpu.VMEM((16, x4.shape[1]), jnp.float32)],
    )(x4, onehot, Wf1, bf1, Wf2, bf2)


def _dgcnn(pos, batch, edge_index, params, *, N, E, NW=32, NBPW=16, RB=98,
           CAPW=20480, CAPB=4096, CE=None, BN=None,
           g_of=None, sr_of=None, interpret=False):
    if CE is None:
        CE = E // 35
    if BN is None:
        BN = N // 10
    if g_of is None:
        g_of = {64: 1024, 128: 512, 256: 256, 512: 128}
    if sr_of is None:
        sr_of = {64: 49, 128: 14, 256: 14, 512: 7}
    src = edge_index[0].astype(jnp.int32)
    dst = edge_index[1].astype(jnp.int32)

    phase0 = _make_phase0(E, NW, NBPW, RB, CAPW, CAPB, CE)
    bsrc, bdl, bcnt = phase0(src, dst)

    douts = (64, 128, 256, 512)

    agg = None
    for li, dout in enumerate(douts):
        p = params['p%d' % (li + 1)]
        if li == 0:
            W1, b1, W2, b2 = p
            h = _run_mlp([pos], (W1, b1[None, :], W2, b2[None, :]), N, BN, dout)
        else:
            W1, b1, W2, b2 = p
            h = _run_mlp([pos, agg],
                         (W1[:3], W1[3:], b1[None, :], W2, b2[None, :]),
                         N, BN, dout)
        aggf = _make_agg(NW, NBPW, RB, CAPB, dout, g_of[dout], sr_of[dout])
        agg_pad = aggf(h, bsrc, bdl, bcnt)
        agg = agg_pad[:N]

    onehot = (batch.astype(jnp.int32)[:, None]
              == jnp.arange(16, dtype=jnp.int32)[None, :]).astype(jnp.float32)
    Wf1, bf1, Wf2, bf2 = params['fc']
    return _run_final(agg, onehot, (Wf1, bf1[None, :], Wf2, bf2[None, :]),
                      N, BN)


def kernel(pos, batch, edge_index, params):
    return _dgcnn(pos, batch, edge_index, params,
                  N=pos.shape[0], E=edge_index.shape[1])
